# Initial kernel scaffold; baseline (speedup 1.0000x reference)
import jax
import jax.numpy as jnp
from jax.experimental import pallas as pl


def kernel(set1, set2):
    raise NotImplementedError("write your pallas kernel here")


# speedup vs baseline: 13.4829x; 13.4829x over previous
import jax
import jax.numpy as jnp
from jax.experimental import pallas as pl
from jax.experimental.pallas import tpu as pltpu

_N1 = 2048
_N2 = 2048
_D = 16
_BLK = 256


def _acosh(v):
    return jnp.log(v + jnp.sqrt(v * v - 1.0))


def _hausdorff_kernel(x_ref, y_ref, out_ref):
    y = y_ref[...]
    yn = jnp.sum(y * y, axis=1, keepdims=True).T
    sy = 1.0 - yn

    colmin = jnp.full((1, _N2), jnp.inf, dtype=jnp.float32)
    rowsum = jnp.float32(0.0)
    for j in range(_N1 // _BLK):
        xb = x_ref[j * _BLK:(j + 1) * _BLK, :]
        xn = jnp.sum(xb * xb, axis=1, keepdims=True)
        sx = 1.0 - xn
        g = jax.lax.dot_general(
            xb, y, (((1,), (1,)), ((), ())),
            preferred_element_type=jnp.float32)
        sq = (xn + yn) - (g + g)
        u = 1.0 + (sq + sq) / (sx * sy)
        colmin = jnp.minimum(colmin, jnp.min(u, axis=0, keepdims=True))
        rmin = jnp.min(u, axis=1, keepdims=True)
        rowsum = rowsum + jnp.sum(_acosh(rmin))
    total = rowsum / _N1 + jnp.sum(_acosh(colmin)) / _N2
    out_ref[...] = jnp.reshape(total, (1, 1))


def kernel(set1, set2):
    out = pl.pallas_call(
        _hausdorff_kernel,
        out_shape=jax.ShapeDtypeStruct((1, 1), jnp.float32),
        in_specs=[
            pl.BlockSpec(memory_space=pltpu.VMEM),
            pl.BlockSpec(memory_space=pltpu.VMEM),
        ],
        out_specs=pl.BlockSpec(memory_space=pltpu.VMEM),
    )(set1, set2)
    return out[0, 0]


# speedup vs baseline: 19.9845x; 1.4822x over previous
import jax
import jax.numpy as jnp
from jax.experimental import pallas as pl
from jax.experimental.pallas import tpu as pltpu

_N1 = 2048
_N2 = 2048
_D = 16


def _acosh(v):
    return jnp.log(v + jnp.sqrt(v * v - 1.0))


def _hausdorff_kernel(x_ref, y_ref, out_ref):
    x = x_ref[...]
    y = y_ref[...]
    xn = jnp.sum(x * x, axis=1, keepdims=True)
    yn = jnp.sum(y * y, axis=1, keepdims=True)
    c = 2.0 / (1.0 - xn)
    b = 1.0 / (1.0 - yn)
    ax = jnp.concatenate([x * (-2.0 * c), xn * c, c], axis=1)
    ay = jnp.concatenate([y * b, b, yn * b], axis=1)
    m = jax.lax.dot_general(
        ax, ay, (((1,), (1,)), ((), ())),
        preferred_element_type=jnp.float32)
    rmin = 1.0 + jnp.min(m, axis=1, keepdims=True)
    cmin = 1.0 + jnp.min(m, axis=0, keepdims=True)
    total = jnp.sum(_acosh(rmin)) / _N1 + jnp.sum(_acosh(cmin)) / _N2
    out_ref[...] = jnp.reshape(total, (1, 1))


def kernel(set1, set2):
    out = pl.pallas_call(
        _hausdorff_kernel,
        out_shape=jax.ShapeDtypeStruct((1, 1), jnp.float32),
        in_specs=[
            pl.BlockSpec(memory_space=pltpu.VMEM),
            pl.BlockSpec(memory_space=pltpu.VMEM),
        ],
        out_specs=pl.BlockSpec(memory_space=pltpu.VMEM),
    )(set1, set2)
    return out[0, 0]
